# baseline (device time: 15196 ns/iter reference)
import functools

import jax
import jax.numpy as jnp
from jax import lax
from jax.experimental import pallas as pl
from jax.experimental.pallas import tpu as pltpu

M = 1024
D = 512
HALF = M // 2
QUART = M // 4


def kernel(partial, gamma):
    p = partial.reshape(M, D)
    g = gamma.reshape(1, D)

    def body(p_ref, g_ref, out_ref,
             sendy_buf, recvy_buf, sumx_buf, recvx_buf,
             sems):
        my_x = lax.axis_index("x")
        my_y = lax.axis_index("y")
        ypeer = (my_x, 1 - my_y)
        xpeer = (1 - my_x, my_y)

        barrier_sem = pltpu.get_barrier_semaphore()
        for nbr in (ypeer, xpeer):
            pl.semaphore_signal(
                barrier_sem, inc=1,
                device_id=nbr, device_id_type=pl.DeviceIdType.MESH,
            )
        pl.semaphore_wait(barrier_sem, 2)

        send_start = (1 - my_y) * HALF + my_x * QUART
        sendy_buf[...] = p_ref[pl.ds(send_start, QUART), :].astype(jnp.bfloat16)
        rdma1 = pltpu.make_async_remote_copy(
            src_ref=sendy_buf,
            dst_ref=recvy_buf,
            send_sem=sems.at[0],
            recv_sem=sems.at[1],
            device_id=ypeer,
            device_id_type=pl.DeviceIdType.MESH,
        )
        rdma1.start()
        rdma1.wait()

        my_start = my_y * HALF + my_x * QUART
        s = (p_ref[pl.ds(my_start, QUART), :]
             + recvy_buf[...].astype(jnp.float32))
        sumx_buf[...] = s.astype(jnp.bfloat16)
        rdma2 = pltpu.make_async_remote_copy(
            src_ref=sumx_buf,
            dst_ref=recvx_buf,
            send_sem=sems.at[2],
            recv_sem=sems.at[3],
            device_id=xpeer,
            device_id_type=pl.DeviceIdType.MESH,
        )
        rdma2.start()

        gamma_row = g_ref[...].astype(jnp.float32)
        rms = jnp.sqrt(jnp.mean(s * s, axis=-1, keepdims=True) + 1e-6)
        out_ref[pl.ds(my_x * QUART, QUART), :] = s / rms * gamma_row

        rdma2.wait()
        s2 = recvx_buf[...].astype(jnp.float32)
        rms2 = jnp.sqrt(jnp.mean(s2 * s2, axis=-1, keepdims=True) + 1e-6)
        out_ref[pl.ds((1 - my_x) * QUART, QUART), :] = s2 / rms2 * gamma_row

    return pl.pallas_call(
        body,
        out_shape=jax.ShapeDtypeStruct((HALF, D), jnp.float32),
        in_specs=[
            pl.BlockSpec(memory_space=pltpu.VMEM),
            pl.BlockSpec(memory_space=pltpu.VMEM),
        ],
        out_specs=pl.BlockSpec(memory_space=pltpu.VMEM),
        scratch_shapes=[
            pltpu.VMEM((QUART, D), jnp.bfloat16),
            pltpu.VMEM((QUART, D), jnp.bfloat16),
            pltpu.VMEM((QUART, D), jnp.bfloat16),
            pltpu.VMEM((QUART, D), jnp.bfloat16),
            pltpu.SemaphoreType.DMA((4,)),
        ],
        compiler_params=pltpu.CompilerParams(collective_id=0),
    )(p, g)


# device time: 13193 ns/iter; 1.1518x vs baseline; 1.1518x over previous
import functools

import jax
import jax.numpy as jnp
from jax import lax
from jax.experimental import pallas as pl
from jax.experimental.pallas import tpu as pltpu

M = 1024
D = 512
HALF = M // 2
QUART = M // 4
N_CHUNK = 4
C = QUART // N_CHUNK


def kernel(partial, gamma):
    p = partial.reshape(M, D)
    g = gamma.reshape(1, D)

    def body(p_ref, g_ref, out_ref,
             sendy_buf, recvy_buf, sumx_buf, recvx_buf,
             y_send, y_recv, x_send, x_recv):
        my_x = lax.axis_index("x")
        my_y = lax.axis_index("y")
        ypeer = (my_x, 1 - my_y)
        xpeer = (1 - my_x, my_y)

        barrier_sem = pltpu.get_barrier_semaphore()
        for nbr in (ypeer, xpeer):
            pl.semaphore_signal(
                barrier_sem, inc=1,
                device_id=nbr, device_id_type=pl.DeviceIdType.MESH,
            )
        pl.semaphore_wait(barrier_sem, 2)

        send_start = (1 - my_y) * HALF + my_x * QUART
        sendy_buf[...] = p_ref[pl.ds(send_start, QUART), :].astype(jnp.bfloat16)
        rdma1 = [
            pltpu.make_async_remote_copy(
                src_ref=sendy_buf.at[pl.ds(k * C, C)],
                dst_ref=recvy_buf.at[pl.ds(k * C, C)],
                send_sem=y_send.at[k],
                recv_sem=y_recv.at[k],
                device_id=ypeer,
                device_id_type=pl.DeviceIdType.MESH,
            )
            for k in range(N_CHUNK)
        ]
        for r in rdma1:
            r.start()

        my_start = my_y * HALF + my_x * QUART
        rdma2 = []
        for k in range(N_CHUNK):
            rdma1[k].wait_recv()
            s_k = (p_ref[pl.ds(my_start + k * C, C), :]
                   + recvy_buf[pl.ds(k * C, C), :].astype(jnp.float32))
            sumx_buf[pl.ds(k * C, C), :] = s_k.astype(jnp.bfloat16)
            r2 = pltpu.make_async_remote_copy(
                src_ref=sumx_buf.at[pl.ds(k * C, C)],
                dst_ref=recvx_buf.at[pl.ds(k * C, C)],
                send_sem=x_send.at[k],
                recv_sem=x_recv.at[k],
                device_id=xpeer,
                device_id_type=pl.DeviceIdType.MESH,
            )
            r2.start()
            rdma2.append(r2)

        gamma_row = g_ref[...].astype(jnp.float32)
        s = sumx_buf[...].astype(jnp.float32)
        rms = jnp.sqrt(jnp.mean(s * s, axis=-1, keepdims=True) + 1e-6)
        out_ref[pl.ds(my_x * QUART, QUART), :] = s / rms * gamma_row

        for r in rdma2:
            r.wait_recv()
        s2 = recvx_buf[...].astype(jnp.float32)
        rms2 = jnp.sqrt(jnp.mean(s2 * s2, axis=-1, keepdims=True) + 1e-6)
        out_ref[pl.ds((1 - my_x) * QUART, QUART), :] = s2 / rms2 * gamma_row

        for r in rdma1:
            r.wait_send()
        for r in rdma2:
            r.wait_send()

    return pl.pallas_call(
        body,
        out_shape=jax.ShapeDtypeStruct((HALF, D), jnp.float32),
        in_specs=[
            pl.BlockSpec(memory_space=pltpu.VMEM),
            pl.BlockSpec(memory_space=pltpu.VMEM),
        ],
        out_specs=pl.BlockSpec(memory_space=pltpu.VMEM),
        scratch_shapes=[
            pltpu.VMEM((QUART, D), jnp.bfloat16),
            pltpu.VMEM((QUART, D), jnp.bfloat16),
            pltpu.VMEM((QUART, D), jnp.bfloat16),
            pltpu.VMEM((QUART, D), jnp.bfloat16),
            pltpu.SemaphoreType.DMA((N_CHUNK,)),
            pltpu.SemaphoreType.DMA((N_CHUNK,)),
            pltpu.SemaphoreType.DMA((N_CHUNK,)),
            pltpu.SemaphoreType.DMA((N_CHUNK,)),
        ],
        compiler_params=pltpu.CompilerParams(collective_id=0),
    )(p, g)


# device time: 13164 ns/iter; 1.1544x vs baseline; 1.0022x over previous
import functools

import jax
import jax.numpy as jnp
from jax import lax
from jax.experimental import pallas as pl
from jax.experimental.pallas import tpu as pltpu

M = 1024
D = 512
HALF = M // 2
QUART = M // 4
N_CHUNK = 4
C = QUART // N_CHUNK


def kernel(partial, gamma):
    def body(p3_ref, g_ref, out_ref,
             sendy_buf, recvy_buf, sumx_buf, recvx_buf,
             y_send, y_recv, x_send, x_recv):
        p_ref = p3_ref.at[0]
        my_x = lax.axis_index("x")
        my_y = lax.axis_index("y")
        ypeer = (my_x, 1 - my_y)
        xpeer = (1 - my_x, my_y)

        barrier_sem = pltpu.get_barrier_semaphore()
        for nbr in (ypeer, xpeer):
            pl.semaphore_signal(
                barrier_sem, inc=1,
                device_id=nbr, device_id_type=pl.DeviceIdType.MESH,
            )
        pl.semaphore_wait(barrier_sem, 2)

        send_start = (1 - my_y) * HALF + my_x * QUART
        sendy_buf[...] = p_ref[pl.ds(send_start, QUART), :].astype(jnp.bfloat16)
        rdma1 = [
            pltpu.make_async_remote_copy(
                src_ref=sendy_buf.at[pl.ds(k * C, C)],
                dst_ref=recvy_buf.at[pl.ds(k * C, C)],
                send_sem=y_send.at[k],
                recv_sem=y_recv.at[k],
                device_id=ypeer,
                device_id_type=pl.DeviceIdType.MESH,
            )
            for k in range(N_CHUNK)
        ]
        for r in rdma1:
            r.start()

        my_start = my_y * HALF + my_x * QUART
        rdma2 = []
        for k in range(N_CHUNK):
            rdma1[k].wait_recv()
            s_k = (p_ref[pl.ds(my_start + k * C, C), :]
                   + recvy_buf[pl.ds(k * C, C), :].astype(jnp.float32))
            sumx_buf[pl.ds(k * C, C), :] = s_k.astype(jnp.bfloat16)
            r2 = pltpu.make_async_remote_copy(
                src_ref=sumx_buf.at[pl.ds(k * C, C)],
                dst_ref=recvx_buf.at[pl.ds(k * C, C)],
                send_sem=x_send.at[k],
                recv_sem=x_recv.at[k],
                device_id=xpeer,
                device_id_type=pl.DeviceIdType.MESH,
            )
            r2.start()
            rdma2.append(r2)

        gamma_row = g_ref[...].astype(jnp.float32).reshape(1, D)
        s = sumx_buf[...].astype(jnp.float32)
        rms = jnp.sqrt(jnp.mean(s * s, axis=-1, keepdims=True) + 1e-6)
        out_ref[pl.ds(my_x * QUART, QUART), :] = s / rms * gamma_row

        for r in rdma2:
            r.wait_recv()
        s2 = recvx_buf[...].astype(jnp.float32)
        rms2 = jnp.sqrt(jnp.mean(s2 * s2, axis=-1, keepdims=True) + 1e-6)
        out_ref[pl.ds((1 - my_x) * QUART, QUART), :] = s2 / rms2 * gamma_row

        for r in rdma1:
            r.wait_send()
        for r in rdma2:
            r.wait_send()

    return pl.pallas_call(
        body,
        out_shape=jax.ShapeDtypeStruct((HALF, D), jnp.float32),
        in_specs=[
            pl.BlockSpec(memory_space=pltpu.VMEM),
            pl.BlockSpec(memory_space=pltpu.VMEM),
        ],
        out_specs=pl.BlockSpec(memory_space=pltpu.VMEM),
        scratch_shapes=[
            pltpu.VMEM((QUART, D), jnp.bfloat16),
            pltpu.VMEM((QUART, D), jnp.bfloat16),
            pltpu.VMEM((QUART, D), jnp.bfloat16),
            pltpu.VMEM((QUART, D), jnp.bfloat16),
            pltpu.SemaphoreType.DMA((N_CHUNK,)),
            pltpu.SemaphoreType.DMA((N_CHUNK,)),
            pltpu.SemaphoreType.DMA((N_CHUNK,)),
            pltpu.SemaphoreType.DMA((N_CHUNK,)),
        ],
        compiler_params=pltpu.CompilerParams(collective_id=0),
    )(partial, gamma)


# device time: 12907 ns/iter; 1.1773x vs baseline; 1.0199x over previous
import contextlib
import functools
import os

import jax
import jax.numpy as jnp
from jax import lax
from jax.experimental import pallas as pl
from jax.experimental.pallas import tpu as pltpu

M = 1024
D = 512
HALF = M // 2
QUART = M // 4
N_CHUNK = 8
C = QUART // N_CHUNK

_SCOPED = os.environ.get("KERNEL_SCOPES") == "1"


def _scope(name):
    return jax.named_scope(name) if _SCOPED else contextlib.nullcontext()


def kernel(partial, gamma):
    def body(p3_ref, g_ref, out_ref,
             sendy_buf, recvy_buf, sumx_buf, recvx_buf,
             y_send, y_recv, x_send, x_recv):
        p_ref = p3_ref.at[0]
        my_x = lax.axis_index("x")
        my_y = lax.axis_index("y")
        ypeer = (my_x, 1 - my_y)
        xpeer = (1 - my_x, my_y)

        with _scope("cast"):
            send_start = (1 - my_y) * HALF + my_x * QUART
            sendy_buf[...] = p_ref[pl.ds(send_start, QUART), :].astype(
                jnp.bfloat16)

        with _scope("barrier"):
            barrier_sem = pltpu.get_barrier_semaphore()
            for nbr in (ypeer, xpeer):
                pl.semaphore_signal(
                    barrier_sem, inc=1,
                    device_id=nbr, device_id_type=pl.DeviceIdType.MESH,
                )
            pl.semaphore_wait(barrier_sem, 2)

        with _scope("send_y"):
            rdma1 = [
                pltpu.make_async_remote_copy(
                    src_ref=sendy_buf.at[pl.ds(k * C, C)],
                    dst_ref=recvy_buf.at[pl.ds(k * C, C)],
                    send_sem=y_send.at[k],
                    recv_sem=y_recv.at[k],
                    device_id=ypeer,
                    device_id_type=pl.DeviceIdType.MESH,
                )
                for k in range(N_CHUNK)
            ]
            for r in rdma1:
                r.start()

        gamma_row = g_ref[...].astype(jnp.float32).reshape(1, D)
        my_start = my_y * HALF + my_x * QUART
        rdma2 = []
        for k in range(N_CHUNK):
            with _scope(f"hop#k={k}"):
                rdma1[k].wait_recv()
                s_k = (p_ref[pl.ds(my_start + k * C, C), :]
                       + recvy_buf[pl.ds(k * C, C), :].astype(jnp.float32))
                rms = jnp.sqrt(
                    jnp.mean(s_k * s_k, axis=-1, keepdims=True) + 1e-6)
                n_k = s_k / rms * gamma_row
                out_ref[pl.ds(my_x * QUART + k * C, C), :] = n_k
                sumx_buf[pl.ds(k * C, C), :] = n_k.astype(jnp.bfloat16)
                r2 = pltpu.make_async_remote_copy(
                    src_ref=sumx_buf.at[pl.ds(k * C, C)],
                    dst_ref=recvx_buf.at[pl.ds(k * C, C)],
                    send_sem=x_send.at[k],
                    recv_sem=x_recv.at[k],
                    device_id=xpeer,
                    device_id_type=pl.DeviceIdType.MESH,
                )
                r2.start()
                rdma2.append(r2)

        with _scope("wait_x"):
            for r in rdma2:
                r.wait_recv()
        with _scope("store_peer"):
            out_ref[pl.ds((1 - my_x) * QUART, QUART), :] = (
                recvx_buf[...].astype(jnp.float32))

        with _scope("drain"):
            for r in rdma1:
                r.wait_send()
            for r in rdma2:
                r.wait_send()

    return pl.pallas_call(
        body,
        out_shape=jax.ShapeDtypeStruct((HALF, D), jnp.float32),
        in_specs=[
            pl.BlockSpec(memory_space=pltpu.VMEM),
            pl.BlockSpec(memory_space=pltpu.VMEM),
        ],
        out_specs=pl.BlockSpec(memory_space=pltpu.VMEM),
        scratch_shapes=[
            pltpu.VMEM((QUART, D), jnp.bfloat16),
            pltpu.VMEM((QUART, D), jnp.bfloat16),
            pltpu.VMEM((QUART, D), jnp.bfloat16),
            pltpu.VMEM((QUART, D), jnp.bfloat16),
            pltpu.SemaphoreType.DMA((N_CHUNK,)),
            pltpu.SemaphoreType.DMA((N_CHUNK,)),
            pltpu.SemaphoreType.DMA((N_CHUNK,)),
            pltpu.SemaphoreType.DMA((N_CHUNK,)),
        ],
        compiler_params=pltpu.CompilerParams(collective_id=0),
    )(partial, gamma)
